# baseline (speedup 1.0000x reference)
import functools

import jax
import jax.numpy as jnp
from jax import lax
from jax.experimental import pallas as pl
from jax.experimental.pallas import tpu as pltpu
from jax.experimental.pallas import tpu_sc as plsc

T = 4096
H = 1024
H2 = H // 2
ID = 1024
E = 7
EP = 8
BT = 256
NB = T // BT + 8
SLOTS = NB * BT
TM = 512

NC = 2
NS = 16
NW = NC * NS
PER_W = SLOTS // NW
CH = 64
NCH = PER_W // CH


def _gelu(t):
    return 0.5 * t * (1.0 + lax.erf(t * 0.7071067811865476))


def _dotT(a, b):
    return lax.dot_general(a, b, (((1,), (1,)), ((), ())),
                           preferred_element_type=jnp.float32)


def _pack(a16):
    lo = lax.bitcast_convert_type(a16[:, :H2], jnp.uint16).astype(jnp.uint32)
    hi = lax.bitcast_convert_type(a16[:, H2:], jnp.uint16).astype(jnp.uint32)
    return lax.bitcast_convert_type(lo | (hi << 16), jnp.float32)


def _unpack(p32):
    u = lax.bitcast_convert_type(p32, jnp.uint32)
    lo = lax.bitcast_convert_type((u & 0xFFFF).astype(jnp.uint16), jnp.bfloat16)
    hi = lax.bitcast_convert_type((u >> 16).astype(jnp.uint16), jnp.bfloat16)
    return jnp.concatenate([lo, hi], axis=1)


def _cumsum_rows(c):
    sh = 1
    while sh < TM:
        c = c + jnp.concatenate(
            [jnp.zeros((sh, EP), c.dtype), c[:TM - sh]], axis=0)
        sh *= 2
    return c


def _router_body(x_ref, rw_ref, rb_ref,
                 xp_ref, gate_ref, eidx_ref, rank_ref, counts_ref, base_ref):
    i = pl.program_id(0)

    @pl.when(i == 0)
    def _():
        base_ref[...] = jnp.zeros((1, EP), jnp.int32)

    xb = x_ref[...]
    xp_ref[...] = _pack(xb.astype(jnp.bfloat16))
    logits = _dotT(xb, rw_ref[...]) + rb_ref[...]
    lanes = lax.broadcasted_iota(jnp.int32, logits.shape, 1)
    logits = jnp.where(lanes < E, logits, -1e30)
    m = jnp.max(logits, axis=1, keepdims=True)
    denom = jnp.sum(jnp.exp(logits - m), axis=1, keepdims=True)
    g = 1.0 / denom
    idx = jnp.min(jnp.where(logits == m, lanes, EP), axis=1, keepdims=True)
    gate_ref[...] = jnp.broadcast_to(g, gate_ref.shape)
    eidx_ref[...] = jnp.broadcast_to(idx, eidx_ref.shape).astype(jnp.int32)
    oh = (jnp.broadcast_to(idx, (TM, EP)) == lanes).astype(jnp.int32)
    csum = _cumsum_rows(oh)
    base = base_ref[...]
    rank = jnp.sum(oh * (csum - 1 + base), axis=1, keepdims=True)
    rank_ref[...] = jnp.broadcast_to(rank, (TM, EP))
    newbase = base + csum[TM - 1:TM, :]
    base_ref[...] = newbase
    counts_ref[...] = newbase


def _router(xf, rWp, rbp):
    return pl.pallas_call(
        _router_body,
        grid=(T // TM,),
        in_specs=[
            pl.BlockSpec((TM, H), lambda i: (i, 0)),
            pl.BlockSpec((EP, H), lambda i: (0, 0)),
            pl.BlockSpec((1, EP), lambda i: (0, 0)),
        ],
        out_specs=[
            pl.BlockSpec((TM, H2), lambda i: (i, 0)),
            pl.BlockSpec((TM, EP), lambda i: (i, 0)),
            pl.BlockSpec((TM, EP), lambda i: (i, 0)),
            pl.BlockSpec((TM, EP), lambda i: (i, 0)),
            pl.BlockSpec((1, EP), lambda i: (0, 0)),
        ],
        out_shape=[
            jax.ShapeDtypeStruct((T, H2), jnp.float32),
            jax.ShapeDtypeStruct((T, EP), jnp.float32),
            jax.ShapeDtypeStruct((T, EP), jnp.int32),
            jax.ShapeDtypeStruct((T, EP), jnp.int32),
            jax.ShapeDtypeStruct((1, EP), jnp.int32),
        ],
        scratch_shapes=[pltpu.VMEM((1, EP), jnp.int32)],
    )(xf, rWp, rbp)


def _place_body(e_ref, r_ref, g_ref, ps_ref, toks_ref, gcols_ref, slot8_ref):
    i = pl.program_id(0)

    @pl.when(i == 0)
    def _():
        toks_ref[...] = jnp.full((SLOTS, 1), T - 1, jnp.int32)
        gcols_ref[...] = jnp.zeros((SLOTS, EP), jnp.float32)

    e = e_ref[...]
    lanes = lax.broadcasted_iota(jnp.int32, (TM, EP), 1)
    ps = jnp.broadcast_to(ps_ref[...], (TM, EP))
    slot = (jnp.sum(jnp.where(e == lanes, ps, 0), axis=1, keepdims=True)
            + r_ref[...][:, :1])
    slot8_ref[...] = jnp.broadcast_to(slot, (TM, EP))

    def body(j, _):
        s = slot8_ref[j, 0]
        toks_ref[pl.ds(s, 1), :] = jnp.reshape(i * TM + j, (1, 1)).astype(jnp.int32)
        gcols_ref[pl.ds(s, 1), :] = jnp.broadcast_to(
            jnp.reshape(g_ref[j, 0], (1, 1)), (1, EP))
        return 0

    lax.fori_loop(0, TM, body, 0, unroll=8)


def _place(eidx8, rank8, gate8, pstart8):
    return pl.pallas_call(
        _place_body,
        grid=(T // TM,),
        in_specs=[
            pl.BlockSpec((TM, EP), lambda i: (i, 0)),
            pl.BlockSpec((TM, EP), lambda i: (i, 0)),
            pl.BlockSpec((TM, EP), lambda i: (i, 0)),
            pl.BlockSpec((1, EP), lambda i: (0, 0)),
        ],
        out_specs=[
            pl.BlockSpec((SLOTS, 1), lambda i: (0, 0)),
            pl.BlockSpec((SLOTS, EP), lambda i: (0, 0)),
            pl.BlockSpec((TM, EP), lambda i: (i, 0)),
        ],
        out_shape=[
            jax.ShapeDtypeStruct((SLOTS, 1), jnp.int32),
            jax.ShapeDtypeStruct((SLOTS, EP), jnp.float32),
            jax.ShapeDtypeStruct((T, EP), jnp.int32),
        ],
    )(eidx8, rank8, gate8, pstart8)


def _shared_body(x_ref, w1_ref, b1_ref, w2_ref, b2_ref, y0_ref):
    xb = x_ref[...]
    xb16 = xb.astype(jnp.bfloat16)
    h = _gelu(_dotT(xb16, w1_ref[...]) + b1_ref[...])
    y0 = _dotT(h.astype(jnp.bfloat16), w2_ref[...]) + b2_ref[...]
    y0_ref[...] = y0 + xb


def _shared(xf, sW1, sb1, sW2, sb2):
    return pl.pallas_call(
        _shared_body,
        grid=(T // TM,),
        in_specs=[
            pl.BlockSpec((TM, H), lambda i: (i, 0)),
            pl.BlockSpec((ID, H), lambda i: (0, 0)),
            pl.BlockSpec((1, ID), lambda i: (0, 0)),
            pl.BlockSpec((H, ID), lambda i: (0, 0)),
            pl.BlockSpec((1, H), lambda i: (0, 0)),
        ],
        out_specs=pl.BlockSpec((TM, H), lambda i: (i, 0)),
        out_shape=jax.ShapeDtypeStruct((T, H), jnp.float32),
    )(xf, sW1, sb1, sW2, sb2)


def _expert_ffn_body(beids_ref, xp_ref, g_ref,
                     w1_ref, b1_ref, w2_ref, b2_ref, out_ref):
    xb16 = _unpack(xp_ref[...])
    h = _gelu(_dotT(xb16, w1_ref[0]) + b1_ref[0])
    y = _dotT(h.astype(jnp.bfloat16), w2_ref[0]) + b2_ref[0]
    out_ref[...] = _pack((g_ref[:, :1] * y).astype(jnp.bfloat16))


def _expert_ffn(beids, Xp32, gcols, rW1, rb1r, rW2, rb2r):
    grid_spec = pltpu.PrefetchScalarGridSpec(
        num_scalar_prefetch=1,
        grid=(NB,),
        in_specs=[
            pl.BlockSpec((BT, H2), lambda i, beids: (i, 0)),
            pl.BlockSpec((BT, EP), lambda i, beids: (i, 0)),
            pl.BlockSpec((1, ID, H), lambda i, beids: (beids[i], 0, 0)),
            pl.BlockSpec((1, 1, ID), lambda i, beids: (beids[i], 0, 0)),
            pl.BlockSpec((1, H, ID), lambda i, beids: (beids[i], 0, 0)),
            pl.BlockSpec((1, 1, H), lambda i, beids: (beids[i], 0, 0)),
        ],
        out_specs=pl.BlockSpec((BT, H2), lambda i, beids: (i, 0)),
    )
    return pl.pallas_call(
        _expert_ffn_body,
        grid_spec=grid_spec,
        out_shape=jax.ShapeDtypeStruct((SLOTS, H2), jnp.float32),
    )(beids, Xp32, gcols, rW1, rb1r, rW2, rb2r)


def _combine_body(slot_ref, y0_ref, r32_ref, out_ref, scr_ref):
    i = pl.program_id(0)

    def body(j, _):
        scr_ref[pl.ds(j, 1), :] = r32_ref[pl.ds(slot_ref[i * TM + j], 1), :]
        return 0

    lax.fori_loop(0, TM, body, 0, unroll=8)
    out_ref[...] = y0_ref[...] + _unpack(scr_ref[...]).astype(jnp.float32)


def _combine(slot, y0, Yp32):
    grid_spec = pltpu.PrefetchScalarGridSpec(
        num_scalar_prefetch=1,
        grid=(T // TM,),
        in_specs=[
            pl.BlockSpec((TM, H), lambda i, slot: (i, 0)),
            pl.BlockSpec((SLOTS, H2), lambda i, slot: (0, 0)),
        ],
        out_specs=pl.BlockSpec((TM, H), lambda i, slot: (i, 0)),
        scratch_shapes=[pltpu.VMEM((TM, H2), jnp.float32)],
    )
    return pl.pallas_call(
        _combine_body,
        grid_spec=grid_spec,
        out_shape=jax.ShapeDtypeStruct((T, H), jnp.float32),
    )(slot, y0, Yp32)


def _sc_gather(x32, toks_g):
    mesh = plsc.VectorSubcoreMesh(core_axis_name="c", subcore_axis_name="s")

    @functools.partial(
        pl.kernel,
        mesh=mesh,
        out_type=jax.ShapeDtypeStruct((SLOTS, H2), jnp.float32),
        scratch_types=[pltpu.VMEM((PER_W,), jnp.int32)]
                      + [pltpu.VMEM((CH, H2), jnp.float32)] * NCH
                      + [pltpu.SemaphoreType.DMA, pltpu.SemaphoreType.DMA],
    )
    def gk(x_hbm, toks_hbm, xp_hbm, idx_v, b0, b1, b2, semg, semw):
        wid = lax.axis_index("s") * NC + lax.axis_index("c")
        base_w = pl.multiple_of(wid * PER_W, 8)
        pltpu.sync_copy(toks_hbm.at[pl.ds(base_w, PER_W)], idx_v)
        bufs = (b0, b1, b2)
        prev = None
        for ch in range(NCH):
            g = pltpu.async_copy(
                x_hbm.at[idx_v.at[pl.ds(ch * CH, CH)]], bufs[ch], semg)
            if prev is not None:
                prev.wait()
            g.wait()
            prev = pltpu.async_copy(
                bufs[ch], xp_hbm.at[pl.ds(base_w + ch * CH, CH)], semw)
        prev.wait()

    return gk(x32, toks_g)


def kernel(x, router_W, router_b, sW1, sb1, sW2, sb2, rW1, rb1, rW2, rb2):
    Bb, S, _ = x.shape
    xf = x.reshape(T, H)
    rWp = jnp.zeros((EP, H), jnp.float32).at[:E].set(router_W)
    rbp = jnp.zeros((1, EP), jnp.float32).at[0, :E].set(router_b)

    xp32, gate8, eidx8, rank8, counts8 = _router(xf, rWp, rbp)
    y0 = _shared(xf, sW1.astype(jnp.bfloat16), sb1.reshape(1, ID),
                 sW2.astype(jnp.bfloat16), sb2.reshape(1, H))

    counts = counts8[0, :E]
    nblk = (counts + BT - 1) // BT
    cnb = jnp.cumsum(nblk)
    pstart = (cnb - nblk) * BT
    pstart8 = jnp.zeros((1, EP), jnp.int32).at[0, :E].set(
        pstart.astype(jnp.int32))
    beids = jnp.clip(
        jnp.sum((jnp.arange(NB, dtype=jnp.int32)[:, None]
                 >= cnb[None, :]).astype(jnp.int32), axis=1),
        0, E - 1).astype(jnp.int32)

    toks2, gcols, slot8 = _place(eidx8, rank8, gate8, pstart8)

    Xp32 = _sc_gather(xp32, toks2.reshape(SLOTS))
    Yp32 = _expert_ffn(beids, Xp32, gcols,
                       rW1.astype(jnp.bfloat16), rb1.reshape(E, 1, ID),
                       rW2.astype(jnp.bfloat16), rb2.reshape(E, 1, H))
    out = _combine(slot8[:, 0], y0, Yp32)
    return out.reshape(Bb, S, H)


# speedup vs baseline: 1.0734x; 1.0734x over previous
import functools

import jax
import jax.numpy as jnp
from jax import lax
from jax.experimental import pallas as pl
from jax.experimental.pallas import tpu as pltpu
from jax.experimental.pallas import tpu_sc as plsc

T = 4096
H = 1024
H2 = H // 2
ID = 1024
E = 7
EP = 8
BT = 256
NB = T // BT + 8
SLOTS = NB * BT
TM = 512

NC = 2
NS = 16
NW = NC * NS
PER_W = SLOTS // NW
CH = 64
NCH = PER_W // CH


def _gelu(t):
    return 0.5 * t * (1.0 + lax.erf(t * 0.7071067811865476))


def _dotT(a, b):
    return lax.dot_general(a, b, (((1,), (1,)), ((), ())),
                           preferred_element_type=jnp.float32)


def _pack(a16):
    lo = lax.bitcast_convert_type(a16[:, :H2], jnp.uint16).astype(jnp.uint32)
    hi = lax.bitcast_convert_type(a16[:, H2:], jnp.uint16).astype(jnp.uint32)
    return lax.bitcast_convert_type(lo | (hi << 16), jnp.float32)


def _unpack(p32):
    u = lax.bitcast_convert_type(p32, jnp.uint32)
    lo = lax.bitcast_convert_type((u & 0xFFFF).astype(jnp.uint16), jnp.bfloat16)
    hi = lax.bitcast_convert_type((u >> 16).astype(jnp.uint16), jnp.bfloat16)
    return jnp.concatenate([lo, hi], axis=1)


def _router_body(x_ref, rw_ref, rb_ref, w1_ref, b1_ref, w2_ref, b2_ref,
                 xp_ref, gate_ref, eidx_ref, y0_ref):
    xb = x_ref[...]
    xb16 = xb.astype(jnp.bfloat16)
    xp_ref[...] = _pack(xb16)
    logits = _dotT(xb, rw_ref[...]) + rb_ref[...]
    lanes = lax.broadcasted_iota(jnp.int32, logits.shape, 1)
    logits = jnp.where(lanes < E, logits, -1e30)
    m = jnp.max(logits, axis=1, keepdims=True)
    denom = jnp.sum(jnp.exp(logits - m), axis=1, keepdims=True)
    g = 1.0 / denom
    idx = jnp.min(jnp.where(logits == m, lanes, EP), axis=1, keepdims=True)
    gate_ref[...] = jnp.broadcast_to(g, gate_ref.shape)
    eidx_ref[...] = jnp.broadcast_to(idx, eidx_ref.shape).astype(jnp.int32)
    h = _gelu(_dotT(xb16, w1_ref[...]) + b1_ref[...])
    y0 = _dotT(h.astype(jnp.bfloat16), w2_ref[...]) + b2_ref[...]
    y0_ref[...] = y0 + xb


def _router(xf, rWp, rbp, sW1, sb1, sW2, sb2):
    return pl.pallas_call(
        _router_body,
        grid=(T // TM,),
        in_specs=[
            pl.BlockSpec((TM, H), lambda i: (i, 0)),
            pl.BlockSpec((EP, H), lambda i: (0, 0)),
            pl.BlockSpec((1, EP), lambda i: (0, 0)),
            pl.BlockSpec((ID, H), lambda i: (0, 0)),
            pl.BlockSpec((1, ID), lambda i: (0, 0)),
            pl.BlockSpec((H, ID), lambda i: (0, 0)),
            pl.BlockSpec((1, H), lambda i: (0, 0)),
        ],
        out_specs=[
            pl.BlockSpec((TM, H2), lambda i: (i, 0)),
            pl.BlockSpec((TM, EP), lambda i: (i, 0)),
            pl.BlockSpec((TM, EP), lambda i: (i, 0)),
            pl.BlockSpec((TM, H), lambda i: (i, 0)),
        ],
        out_shape=[
            jax.ShapeDtypeStruct((T, H2), jnp.float32),
            jax.ShapeDtypeStruct((T, EP), jnp.float32),
            jax.ShapeDtypeStruct((T, EP), jnp.int32),
            jax.ShapeDtypeStruct((T, H), jnp.float32),
        ],
    )(xf, rWp, rbp, sW1, sb1, sW2, sb2)


def _expert_ffn_body(beids_ref, xp_ref, g_ref,
                     w1_ref, b1_ref, w2_ref, b2_ref, out_ref):
    xb16 = _unpack(xp_ref[...])
    h = _gelu(_dotT(xb16, w1_ref[0]) + b1_ref[0])
    y = _dotT(h.astype(jnp.bfloat16), w2_ref[0]) + b2_ref[0]
    out_ref[...] = _pack((g_ref[:, :1] * y).astype(jnp.bfloat16))


def _expert_ffn(beids, Xp32, gcols, rW1, rb1r, rW2, rb2r):
    grid_spec = pltpu.PrefetchScalarGridSpec(
        num_scalar_prefetch=1,
        grid=(NB,),
        in_specs=[
            pl.BlockSpec((BT, H2), lambda i, beids: (i, 0)),
            pl.BlockSpec((BT, EP), lambda i, beids: (i, 0)),
            pl.BlockSpec((1, ID, H), lambda i, beids: (beids[i], 0, 0)),
            pl.BlockSpec((1, 1, ID), lambda i, beids: (beids[i], 0, 0)),
            pl.BlockSpec((1, H, ID), lambda i, beids: (beids[i], 0, 0)),
            pl.BlockSpec((1, 1, H), lambda i, beids: (beids[i], 0, 0)),
        ],
        out_specs=pl.BlockSpec((BT, H2), lambda i, beids: (i, 0)),
    )
    return pl.pallas_call(
        _expert_ffn_body,
        grid_spec=grid_spec,
        out_shape=jax.ShapeDtypeStruct((SLOTS, H2), jnp.float32),
    )(beids, Xp32, gcols, rW1, rb1r, rW2, rb2r)


def _combine_body(slot_ref, y0_ref, r32_ref, out_ref, scr_ref):
    i = pl.program_id(0)

    def body(j, _):
        scr_ref[pl.ds(j, 1), :] = r32_ref[pl.ds(slot_ref[i * TM + j], 1), :]
        return 0

    lax.fori_loop(0, TM, body, 0, unroll=8)
    out_ref[...] = y0_ref[...] + _unpack(scr_ref[...]).astype(jnp.float32)


def _combine(slot, y0, Yp32):
    grid_spec = pltpu.PrefetchScalarGridSpec(
        num_scalar_prefetch=1,
        grid=(T // TM,),
        in_specs=[
            pl.BlockSpec((TM, H), lambda i, slot: (i, 0)),
            pl.BlockSpec((SLOTS, H2), lambda i, slot: (0, 0)),
        ],
        out_specs=pl.BlockSpec((TM, H), lambda i, slot: (i, 0)),
        scratch_shapes=[pltpu.VMEM((TM, H2), jnp.float32)],
    )
    return pl.pallas_call(
        _combine_body,
        grid_spec=grid_spec,
        out_shape=jax.ShapeDtypeStruct((T, H), jnp.float32),
    )(slot, y0, Yp32)


def _sc_gather(x32, toks_g):
    mesh = plsc.VectorSubcoreMesh(core_axis_name="c", subcore_axis_name="s")

    @functools.partial(
        pl.kernel,
        mesh=mesh,
        out_type=jax.ShapeDtypeStruct((SLOTS, H2), jnp.float32),
        scratch_types=[pltpu.VMEM((PER_W,), jnp.int32)]
                      + [pltpu.VMEM((CH, H2), jnp.float32)] * NCH
                      + [pltpu.SemaphoreType.DMA, pltpu.SemaphoreType.DMA],
    )
    def gk(x_hbm, toks_hbm, xp_hbm, idx_v, b0, b1, b2, semg, semw):
        wid = lax.axis_index("s") * NC + lax.axis_index("c")
        base_w = pl.multiple_of(wid * PER_W, 8)
        pltpu.sync_copy(toks_hbm.at[pl.ds(base_w, PER_W)], idx_v)
        bufs = (b0, b1, b2)
        prev = None
        for ch in range(NCH):
            g = pltpu.async_copy(
                x_hbm.at[idx_v.at[pl.ds(ch * CH, CH)]], bufs[ch], semg)
            if prev is not None:
                prev.wait()
            g.wait()
            prev = pltpu.async_copy(
                bufs[ch], xp_hbm.at[pl.ds(base_w + ch * CH, CH)], semw)
        prev.wait()

    return gk(x32, toks_g)


def kernel(x, router_W, router_b, sW1, sb1, sW2, sb2, rW1, rb1, rW2, rb2):
    Bb, S, _ = x.shape
    xf = x.reshape(T, H)
    rWp = jnp.zeros((EP, H), jnp.float32).at[:E].set(router_W)
    rbp = jnp.zeros((1, EP), jnp.float32).at[0, :E].set(router_b)

    xp32, gate8, eidx8, y0 = _router(
        xf, rWp, rbp, sW1.astype(jnp.bfloat16), sb1.reshape(1, ID),
        sW2.astype(jnp.bfloat16), sb2.reshape(1, H))

    eid = eidx8[:, 0]
    gate = gate8[:, 0]

    oh = (eid[:, None] == jnp.arange(E, dtype=jnp.int32)[None, :]).astype(jnp.int32)
    csum = jnp.cumsum(oh, axis=0)
    rank = jnp.sum(csum * oh, axis=1) - 1
    counts = csum[-1]
    nblk = (counts + BT - 1) // BT
    cnb = jnp.cumsum(nblk)
    pstart = (cnb - nblk) * BT
    slot = jnp.sum(oh * pstart[None, :], axis=1) + rank
    toks = jnp.full((SLOTS,), T, jnp.int32).at[slot].set(
        jnp.arange(T, dtype=jnp.int32))
    toks_g = jnp.minimum(toks, T - 1)
    gate_slot = jnp.zeros((SLOTS,), jnp.float32).at[slot].set(gate)
    gcols = jnp.broadcast_to(gate_slot[:, None], (SLOTS, EP))
    beids = jnp.clip(
        jnp.sum((jnp.arange(NB, dtype=jnp.int32)[:, None]
                 >= cnb[None, :]).astype(jnp.int32), axis=1),
        0, E - 1).astype(jnp.int32)

    Xp32 = _sc_gather(xp32, toks_g)
    Yp32 = _expert_ffn(beids, Xp32, gcols,
                       rW1.astype(jnp.bfloat16), rb1.reshape(E, 1, ID),
                       rW2.astype(jnp.bfloat16), rb2.reshape(E, 1, H))
    out = _combine(slot.astype(jnp.int32), y0, Yp32)
    return out.reshape(Bb, S, H)
